# baseline (speedup 1.0000x reference)
import functools

import jax
import jax.numpy as jnp
from jax import lax
from jax.experimental import pallas as pl
from jax.experimental.pallas import tpu as pltpu
from jax.experimental.pallas import tpu_sc as plsc

_LEVELS = 1024
_TS = 64
_D = 10240
_N = 64
_NW = 32
_SPW = _N // _NW
_LANES = 16
_CHUNKS = _D // _LANES
_UNROLL = 4
_RNE = 8388608.0

_SLOTS = ((1, 2, 3, 0), (5, 6, 7, 4))


def _sc_body(inp_hbm, lx_hbm, ly_hbm, lz_hbm, tt_hbm,
             out_hbm, inp_v, idxbuf, x0, y0, z0, t0, x1, y1, z1, t1, acc,
             sem0, sem1):
    sidx = lax.axis_index("s")
    cidx = lax.axis_index("c")
    w = sidx * 2 + cidx
    pltpu.sync_copy(inp_hbm, inp_v)
    v = inp_v[sidx, :]
    col = lax.iota(jnp.int32, 16) & 3
    is_t = col == 0
    low = jnp.where(is_t, 0.0, -5.0).astype(jnp.float32)
    span = jnp.where(is_t, float(_TS), 10.0).astype(jnp.float32)
    scale = jnp.where(is_t, _TS - 1.0, _LEVELS - 1.0).astype(jnp.float32)
    u = jnp.clip((v - low) / span, 0.0, 1.0) * scale
    r = (u + _RNE) - _RNE
    idxbuf[0, pl.ds(0, 16)] = r.astype(jnp.int32)

    off = cidx * 8
    tabs = (lx_hbm, ly_hbm, lz_hbm, tt_hbm)
    waits0 = [
        pltpu.async_copy(tab.at[idxbuf.at[0, pl.ds(off + p, 1)]], r, sem0)
        for p, tab, r in zip(_SLOTS[0], tabs, (x0, y0, z0, t0))
    ]
    waits1 = [
        pltpu.async_copy(tab.at[idxbuf.at[0, pl.ds(off + p, 1)]], r, sem1)
        for p, tab, r in zip(_SLOTS[1], tabs, (x1, y1, z1, t1))
    ]
    for h in waits0:
        h.wait()

    @plsc.parallel_loop(0, _CHUNKS, 1, unroll=_UNROLL)
    def _loop0(i):
        s = pl.ds(i * _LANES, _LANES)
        acc[s] = (x0[0, s] + y0[0, s] + z0[0, s]) * t0[0, s]

    for h in waits1:
        h.wait()

    @plsc.parallel_loop(0, _CHUNKS, 1, unroll=_UNROLL)
    def _loop1(i):
        s = pl.ds(i * _LANES, _LANES)
        acc[s] = acc[s] * ((x1[0, s] + y1[0, s] + z1[0, s]) * t1[0, s])

    pltpu.sync_copy(acc, out_hbm.at[w])


def _sc_partials(inp, lx, ly, lz, tt):
    mesh = plsc.VectorSubcoreMesh(core_axis_name="c", subcore_axis_name="s")
    f = pl.kernel(
        _sc_body,
        out_type=jax.ShapeDtypeStruct((_NW, _D), jnp.float32),
        mesh=mesh,
        scratch_types=[
            pltpu.VMEM((16, 16), jnp.float32),
            pltpu.VMEM((1, 16), jnp.int32),
            pltpu.VMEM((1, _D), jnp.float32),
            pltpu.VMEM((1, _D), jnp.float32),
            pltpu.VMEM((1, _D), jnp.float32),
            pltpu.VMEM((1, _D), jnp.float32),
            pltpu.VMEM((1, _D), jnp.float32),
            pltpu.VMEM((1, _D), jnp.float32),
            pltpu.VMEM((1, _D), jnp.float32),
            pltpu.VMEM((1, _D), jnp.float32),
            pltpu.VMEM((_D,), jnp.float32),
            pltpu.SemaphoreType.DMA,
            pltpu.SemaphoreType.DMA,
        ],
    )
    return f(inp, lx, ly, lz, tt)


def _fb_body(f_ref, w_ref, b_ref, o_ref):
    p = jnp.dot(f_ref[...], w_ref[...], preferred_element_type=jnp.float32)
    o_ref[...] = jnp.cos(p + b_ref[...]) * jnp.sin(p)


def _comb_body(p_ref, fb_ref, o_ref):
    tot = p_ref[0]
    for i in range(1, _NW):
        tot = tot * p_ref[i]
    o_ref[...] = jnp.where(tot * fb_ref[...] > 0, 1.0, -1.0).astype(jnp.float32)


_CB = 5


def _tc_fbind(feat, wt, b):
    return pl.pallas_call(
        _fb_body,
        out_shape=jax.ShapeDtypeStruct((_D,), jnp.float32),
    )(feat, wt, b)


def _tc_combine(partials, fbind):
    blk = _D // _CB
    return pl.pallas_call(
        _comb_body,
        grid=(_CB,),
        in_specs=[
            pl.BlockSpec((_NW, blk), lambda i: (0, i)),
            pl.BlockSpec((blk,), lambda i: (i,)),
        ],
        out_specs=pl.BlockSpec((blk,), lambda i: (i,)),
        out_shape=jax.ShapeDtypeStruct((_D,), jnp.float32),
    )(partials, fbind)


def kernel(input, feat, level_x, level_y, level_z, time_table, W, b):
    partials = _sc_partials(input.reshape(16, 16), level_x, level_y, level_z,
                            time_table)
    fbind = _tc_fbind(feat, W.T, b)
    return _tc_combine(partials, fbind)


# speedup vs baseline: 1.0483x; 1.0483x over previous
import functools

import jax
import jax.numpy as jnp
from jax import lax
from jax.experimental import pallas as pl
from jax.experimental.pallas import tpu as pltpu
from jax.experimental.pallas import tpu_sc as plsc

_LEVELS = 1024
_TS = 64
_D = 10240
_N = 64
_NW = 32
_SPW = _N // _NW
_LANES = 16
_CHUNKS = _D // _LANES
_UNROLL = 4
_RNE = 8388608.0

_SLOTS = ((1, 2, 3, 0), (5, 6, 7, 4))


def _sc_body(inp_hbm, lx_hbm, ly_hbm, lz_hbm, tt_hbm,
             out_hbm, inp_v, idxbuf, x0, y0, z0, t0, x1, y1, z1, t1, acc,
             sem0, sem1):
    sidx = lax.axis_index("s")
    cidx = lax.axis_index("c")
    w = sidx * 2 + cidx
    pltpu.sync_copy(inp_hbm, inp_v)
    v = inp_v[sidx, :]
    col = lax.iota(jnp.int32, 16) & 3
    is_t = col == 0
    low = jnp.where(is_t, 0.0, -5.0).astype(jnp.float32)
    span = jnp.where(is_t, float(_TS), 10.0).astype(jnp.float32)
    scale = jnp.where(is_t, _TS - 1.0, _LEVELS - 1.0).astype(jnp.float32)
    u = jnp.clip((v - low) / span, 0.0, 1.0) * scale
    r = (u + _RNE) - _RNE
    idxbuf[0, pl.ds(0, 16)] = r.astype(jnp.int32)

    off = cidx * 8
    tabs = (lx_hbm, ly_hbm, lz_hbm, tt_hbm)
    waits0 = [
        pltpu.async_copy(tab.at[idxbuf.at[0, pl.ds(off + p, 1)]], r, sem0)
        for p, tab, r in zip(_SLOTS[0], tabs, (x0, y0, z0, t0))
    ]
    waits1 = [
        pltpu.async_copy(tab.at[idxbuf.at[0, pl.ds(off + p, 1)]], r, sem1)
        for p, tab, r in zip(_SLOTS[1], tabs, (x1, y1, z1, t1))
    ]
    for h in waits0:
        h.wait()

    @plsc.parallel_loop(0, _CHUNKS, 1, unroll=_UNROLL)
    def _loop0(i):
        s = pl.ds(i * _LANES, _LANES)
        acc[s] = (x0[0, s] + y0[0, s] + z0[0, s]) * t0[0, s]

    for h in waits1:
        h.wait()

    @plsc.parallel_loop(0, _CHUNKS, 1, unroll=_UNROLL)
    def _loop1(i):
        s = pl.ds(i * _LANES, _LANES)
        acc[s] = acc[s] * ((x1[0, s] + y1[0, s] + z1[0, s]) * t1[0, s])

    pltpu.sync_copy(acc, out_hbm.at[w])


def _sc_partials(inp, lx, ly, lz, tt):
    mesh = plsc.VectorSubcoreMesh(core_axis_name="c", subcore_axis_name="s")
    f = pl.kernel(
        _sc_body,
        out_type=jax.ShapeDtypeStruct((_NW, _D), jnp.float32),
        mesh=mesh,
        scratch_types=[
            pltpu.VMEM((16, 16), jnp.float32),
            pltpu.VMEM((1, 16), jnp.int32),
            pltpu.VMEM((1, _D), jnp.float32),
            pltpu.VMEM((1, _D), jnp.float32),
            pltpu.VMEM((1, _D), jnp.float32),
            pltpu.VMEM((1, _D), jnp.float32),
            pltpu.VMEM((1, _D), jnp.float32),
            pltpu.VMEM((1, _D), jnp.float32),
            pltpu.VMEM((1, _D), jnp.float32),
            pltpu.VMEM((1, _D), jnp.float32),
            pltpu.VMEM((_D,), jnp.float32),
            pltpu.SemaphoreType.DMA,
            pltpu.SemaphoreType.DMA,
        ],
    )
    return f(inp, lx, ly, lz, tt)


def _fb_body(f_ref, w_ref, b_ref, o_ref):
    p = jnp.dot(f_ref[...], w_ref[...], preferred_element_type=jnp.float32)
    o_ref[...] = jnp.cos(p + b_ref[...]) * jnp.sin(p)


def _comb_body(p_ref, fb_ref, o_ref):
    tot = p_ref[0]
    for i in range(1, _NW):
        tot = tot * p_ref[i]
    o_ref[...] = jnp.where(tot * fb_ref[...] > 0, 1.0, -1.0).astype(jnp.float32)


_CB = 5


def _tc_fbind(feat, wt, b):
    return pl.pallas_call(
        _fb_body,
        out_shape=jax.ShapeDtypeStruct((_D,), jnp.float32),
    )(feat, wt, b)


def _tc_combine(partials, fbind):
    return pl.pallas_call(
        _comb_body,
        out_shape=jax.ShapeDtypeStruct((_D,), jnp.float32),
    )(partials, fbind)


def kernel(input, feat, level_x, level_y, level_z, time_table, W, b):
    partials = _sc_partials(input.reshape(16, 16), level_x, level_y, level_z,
                            time_table)
    fbind = _tc_fbind(feat, W.T, b)
    return _tc_combine(partials, fbind)


# speedup vs baseline: 1.1091x; 1.0580x over previous
import functools

import jax
import jax.numpy as jnp
from jax import lax
from jax.experimental import pallas as pl
from jax.experimental.pallas import tpu as pltpu
from jax.experimental.pallas import tpu_sc as plsc

_LEVELS = 1024
_TS = 64
_D = 10240
_N = 64
_NW = 32
_SPW = _N // _NW
_LANES = 16
_CHUNKS = _D // _LANES
_UNROLL = 4

_COLS = (1, 2, 3, 0)


def _sc_body(idx_hbm, lx_hbm, ly_hbm, lz_hbm, tt_hbm,
             out_hbm, idx_v, x0, y0, z0, t0, x1, y1, z1, t1, acc, sem0, sem1):
    w = lax.axis_index("s") * 2 + lax.axis_index("c")
    pltpu.sync_copy(idx_hbm.at[w], idx_v)
    tabs = (lx_hbm, ly_hbm, lz_hbm, tt_hbm)
    waits0 = [
        pltpu.async_copy(tab.at[idx_v.at[0, pl.ds(c, 1)]], r, sem0)
        for c, tab, r in zip(_COLS, tabs, (x0, y0, z0, t0))
    ]
    waits1 = [
        pltpu.async_copy(tab.at[idx_v.at[1, pl.ds(c, 1)]], r, sem1)
        for c, tab, r in zip(_COLS, tabs, (x1, y1, z1, t1))
    ]
    for h in waits0:
        h.wait()

    @plsc.parallel_loop(0, _CHUNKS, 1, unroll=_UNROLL)
    def _loop0(i):
        s = pl.ds(i * _LANES, _LANES)
        acc[s] = (x0[0, s] + y0[0, s] + z0[0, s]) * t0[0, s]

    for h in waits1:
        h.wait()

    @plsc.parallel_loop(0, _CHUNKS, 1, unroll=_UNROLL)
    def _loop1(i):
        s = pl.ds(i * _LANES, _LANES)
        acc[s] = acc[s] * ((x1[0, s] + y1[0, s] + z1[0, s]) * t1[0, s])

    pltpu.sync_copy(acc, out_hbm.at[w])


def _sc_partials(idx, lx, ly, lz, tt):
    mesh = plsc.VectorSubcoreMesh(core_axis_name="c", subcore_axis_name="s")
    f = pl.kernel(
        _sc_body,
        out_type=jax.ShapeDtypeStruct((_NW, _D), jnp.float32),
        mesh=mesh,
        scratch_types=[
            pltpu.VMEM((_SPW, 4), jnp.int32),
            pltpu.VMEM((1, _D), jnp.float32),
            pltpu.VMEM((1, _D), jnp.float32),
            pltpu.VMEM((1, _D), jnp.float32),
            pltpu.VMEM((1, _D), jnp.float32),
            pltpu.VMEM((1, _D), jnp.float32),
            pltpu.VMEM((1, _D), jnp.float32),
            pltpu.VMEM((1, _D), jnp.float32),
            pltpu.VMEM((1, _D), jnp.float32),
            pltpu.VMEM((_D,), jnp.float32),
            pltpu.SemaphoreType.DMA,
            pltpu.SemaphoreType.DMA,
        ],
    )
    return f(idx, lx, ly, lz, tt)


def _fb_body(f_ref, w_ref, b_ref, o_ref):
    p = jnp.dot(f_ref[...], w_ref[...], preferred_element_type=jnp.float32)
    o_ref[...] = jnp.cos(p + b_ref[...]) * jnp.sin(p)


def _comb_body(p_ref, fb_ref, o_ref):
    tot = p_ref[0]
    for i in range(1, _NW):
        tot = tot * p_ref[i]
    o_ref[...] = jnp.where(tot * fb_ref[...] > 0, 1.0, -1.0).astype(jnp.float32)


def _tc_fbind(feat, wt, b):
    return pl.pallas_call(
        _fb_body,
        out_shape=jax.ShapeDtypeStruct((_D,), jnp.float32),
    )(feat, wt, b)


def _tc_combine(partials, fbind):
    return pl.pallas_call(
        _comb_body,
        out_shape=jax.ShapeDtypeStruct((_D,), jnp.float32),
    )(partials, fbind)


def kernel(input, feat, level_x, level_y, level_z, time_table, W, b):
    lows = jnp.array([0.0, -5.0, -5.0, -5.0], dtype=jnp.float32)
    spans = jnp.array([float(_TS), 10.0, 10.0, 10.0], dtype=jnp.float32)
    scales = jnp.array([_TS - 1.0, _LEVELS - 1.0, _LEVELS - 1.0, _LEVELS - 1.0],
                       dtype=jnp.float32)
    idx = jnp.round(
        jnp.clip((input - lows) / spans, 0.0, 1.0) * scales
    ).astype(jnp.int32).reshape(_NW, _SPW, 4)

    partials = _sc_partials(idx, level_x, level_y, level_z, time_table)
    fbind = _tc_fbind(feat, W.T, b)
    return _tc_combine(partials, fbind)
